# Initial kernel scaffold; baseline (speedup 1.0000x reference)
import jax
import jax.numpy as jnp
from jax.experimental import pallas as pl


def kernel(predictions, targets):
    raise NotImplementedError("write your pallas kernel here")


# speedup vs baseline: 11.6979x; 11.6979x over previous
import jax
import jax.numpy as jnp
from jax.experimental import pallas as pl
from jax.experimental.pallas import tpu as pltpu

_N = 8
_C = 19
_H = 512
_W = 512
_K = 4096
_HB = 64
_NHB = _H // _HB


def _ce_topk_kernel(pred_ref, tgt_ref, out_ref, loss_ref):
    n = pl.program_id(0)
    hb = pl.program_id(1)

    x = pred_ref[0]
    tgt = tgt_ref[0]
    m = x[0]
    for c in range(1, _C):
        m = jnp.maximum(m, x[c])
    s = jnp.zeros((_HB, _W), jnp.float32)
    xt = jnp.zeros((_HB, _W), jnp.float32)
    for c in range(_C):
        s = s + jnp.exp(x[c] - m)
        xt = xt + jnp.where(tgt == c, x[c], 0.0)
    nll = m + jnp.log(s) - xt
    loss_ref[pl.ds(hb * _HB, _HB), :] = jnp.maximum(nll, 0.0)

    @pl.when(hb == _NHB - 1)
    def _select():
        loss = loss_ref[...]

        def body(it, prefix):
            cand = prefix | (1 << (30 - it))
            t_f = jax.lax.bitcast_convert_type(cand, jnp.float32)
            cnt = jnp.sum((loss >= t_f).astype(jnp.int32))
            return jnp.where(cnt >= _K, cand, prefix)

        t_bits = jax.lax.fori_loop(0, 31, body, jnp.int32(0))
        t_val = jax.lax.bitcast_convert_type(t_bits, jnp.float32)

        gt = loss > t_val
        sum_gt = jnp.sum(jnp.where(gt, loss, 0.0))
        cnt_gt = jnp.sum(gt.astype(jnp.int32))
        topk_sum = sum_gt + t_val * (_K - cnt_gt).astype(jnp.float32)
        contrib = topk_sum * (1.0 / (_K * _N))

        @pl.when(n == 0)
        def _init():
            out_ref[0, 0] = contrib

        @pl.when(n != 0)
        def _acc():
            out_ref[0, 0] = out_ref[0, 0] + contrib


def kernel(predictions, targets):
    targets = targets.astype(jnp.int32)
    out = pl.pallas_call(
        _ce_topk_kernel,
        grid=(_N, _NHB),
        in_specs=[
            pl.BlockSpec((1, _C, _HB, _W), lambda n, hb: (n, 0, hb, 0)),
            pl.BlockSpec((1, _HB, _W), lambda n, hb: (n, hb, 0)),
        ],
        out_specs=pl.BlockSpec(
            (1, 1), lambda n, hb: (0, 0), memory_space=pltpu.SMEM
        ),
        out_shape=jax.ShapeDtypeStruct((1, 1), jnp.float32),
        scratch_shapes=[pltpu.VMEM((_H, _W), jnp.float32)],
    )(predictions, targets)
    return out[0, 0]
